# Initial kernel scaffold; baseline (speedup 1.0000x reference)
import jax
import jax.numpy as jnp
from jax.experimental import pallas as pl


def kernel(x, edge_index, edge_attr, Wa1, ba1, Wb1, bb1, root1, bias1, Wa2, ba2, Wb2, bb2, root2, bias2, Wfc, bfc):
    raise NotImplementedError("write your pallas kernel here")


# speedup vs baseline: 2.4583x; 2.4583x over previous
import functools

import jax
import jax.numpy as jnp
from jax import lax
from jax.experimental import pallas as pl
from jax.experimental.pallas import tpu as pltpu
from jax.experimental.pallas import tpu_sc as plsc

_N = 10000
_NP = 10112
_E = 160000
_EP = 163840
_DIN = 128
_DE = 16
_HID = 64
_H = 16
_OUT = 16

_CHUNK = 128
_ROWS = _EP // _CHUNK
_EB = 2048


def _sc_dims():
    try:
        info = plsc.get_sparse_core_info()
        return info.num_cores, info.num_subcores
    except Exception:
        return 2, 16


def _gather_rows(table, idx2d, d):
    nc, ns = _sc_dims()
    rpw = _ROWS // (nc * ns)
    mesh = plsc.VectorSubcoreMesh(core_axis_name="c", subcore_axis_name="s")

    @functools.partial(
        pl.kernel,
        out_type=jax.ShapeDtypeStruct((_EP, d), jnp.float32),
        mesh=mesh,
        scratch_types=[
            pltpu.VMEM((rpw, _CHUNK), jnp.int32),
            pltpu.VMEM((_CHUNK, d), jnp.float32),
            pltpu.SemaphoreType.DMA,
        ],
        compiler_params=pltpu.CompilerParams(use_tc_tiling_on_sc=(d >= 128)),
    )
    def k(table_hbm, idx_hbm, out_hbm, idx_v, rows_v, sem):
        w = lax.axis_index("s") * nc + lax.axis_index("c")
        r0 = w * rpw
        pltpu.sync_copy(idx_hbm.at[pl.ds(r0, rpw)], idx_v)

        def body(j, carry):
            pltpu.async_copy(table_hbm.at[idx_v.at[j]], rows_v, sem).wait()
            pltpu.sync_copy(rows_v, out_hbm.at[pl.ds((r0 + j) * _CHUNK, _CHUNK)])
            return carry

        lax.fori_loop(0, rpw, body, 0)

    return k(table, idx2d)


def _scatter_add(msg, dst2d, zeros_n, ones_c, with_counts):
    nc, ns = _sc_dims()
    rpw = _ROWS // (nc * ns)
    nz = _NP // ns
    mesh = plsc.VectorSubcoreMesh(core_axis_name="c", subcore_axis_name="s")

    out_types = [jax.ShapeDtypeStruct((nc, _NP, _H), jnp.float32)]
    scratch = [
        pltpu.VMEM((rpw, _CHUNK), jnp.int32),
        pltpu.VMEM((_CHUNK, _H), jnp.float32),
        pltpu.VMEM_SHARED((_NP, _H), jnp.float32),
    ]
    if with_counts:
        out_types.append(jax.ShapeDtypeStruct((nc, _NP, _H), jnp.float32))
        scratch += [
            pltpu.VMEM((_CHUNK, _H), jnp.float32),
            pltpu.VMEM_SHARED((_NP, _H), jnp.float32),
        ]

    @functools.partial(
        pl.kernel,
        out_type=tuple(out_types),
        mesh=mesh,
        scratch_types=scratch,
        compiler_params=pltpu.CompilerParams(use_tc_tiling_on_sc=False),
    )
    def k(msg_hbm, dst_hbm, zeros_hbm, ones_hbm, *rest):
        if with_counts:
            s_out, c_out, idx_v, mv, s_sh, ones_v, c_sh = rest
        else:
            s_out, idx_v, mv, s_sh = rest
        c = lax.axis_index("c")
        s = lax.axis_index("s")
        w = s * nc + c
        rz = s * nz
        pltpu.sync_copy(zeros_hbm.at[pl.ds(rz, nz)], s_sh.at[pl.ds(rz, nz)])
        if with_counts:
            pltpu.sync_copy(zeros_hbm.at[pl.ds(rz, nz)], c_sh.at[pl.ds(rz, nz)])
            pltpu.sync_copy(ones_hbm, ones_v)
        r0 = w * rpw
        pltpu.sync_copy(dst_hbm.at[pl.ds(r0, rpw)], idx_v)
        plsc.subcore_barrier()

        def body(j, carry):
            pltpu.sync_copy(msg_hbm.at[pl.ds((r0 + j) * _CHUNK, _CHUNK)], mv)
            pltpu.sync_copy(mv, s_sh.at[idx_v.at[j]], add=True)
            if with_counts:
                pltpu.sync_copy(ones_v, c_sh.at[idx_v.at[j]], add=True)
            return carry

        lax.fori_loop(0, rpw, body, 0)

        plsc.subcore_barrier()
        pltpu.sync_copy(s_sh.at[pl.ds(rz, nz)], s_out.at[c, pl.ds(rz, nz)])
        if with_counts:
            pltpu.sync_copy(c_sh.at[pl.ds(rz, nz)], c_out.at[c, pl.ds(rz, nz)])

    return k(msg, dst2d, zeros_n, ones_c)


def _edge_messages(xj, ea, Wa, ba, W2d, bbr, kmat):
    e, din = xj.shape
    grid = e // _EB

    def body(xj_ref, ea_ref, wa_ref, ba_ref, w2d_ref, bbr_ref, k_ref, out_ref):
        h = jax.nn.relu(
            jnp.dot(ea_ref[...], wa_ref[...], preferred_element_type=jnp.float32)
            + ba_ref[...]
        )
        g = jnp.dot(xj_ref[...], w2d_ref[...], preferred_element_type=jnp.float32)
        hh = jnp.concatenate([h] * _H, axis=1)
        out_ref[...] = (
            jnp.dot(g * hh, k_ref[...], preferred_element_type=jnp.float32)
            + jnp.dot(xj_ref[...], bbr_ref[...], preferred_element_type=jnp.float32)
        )

    return pl.pallas_call(
        body,
        grid=(grid,),
        in_specs=[
            pl.BlockSpec((_EB, din), lambda i: (i, 0)),
            pl.BlockSpec((_EB, _DE), lambda i: (i, 0)),
            pl.BlockSpec((_DE, _HID), lambda i: (0, 0)),
            pl.BlockSpec((1, _HID), lambda i: (0, 0)),
            pl.BlockSpec((din, _H * _HID), lambda i: (0, 0)),
            pl.BlockSpec((din, _H), lambda i: (0, 0)),
            pl.BlockSpec((_H * _HID, _H), lambda i: (0, 0)),
        ],
        out_specs=pl.BlockSpec((_EB, _H), lambda i: (i, 0)),
        out_shape=jax.ShapeDtypeStruct((e, _H), jnp.float32),
    )(xj, ea, Wa, ba, W2d, bbr, kmat)


def _node_update(s0, s1, c0, c1, feats, root, bias, wfc=None, bfc=None):
    def body(*refs):
        if wfc is None:
            s0r, s1r, c0r, c1r, fr, rr, br, out = refs
        else:
            s0r, s1r, c0r, c1r, fr, rr, br, wr, bwr, out = refs
        cnt = jnp.maximum(c0r[...] + c1r[...], 1.0)
        h = jax.nn.relu(
            (s0r[...] + s1r[...]) / cnt
            + jnp.dot(fr[...], rr[...], preferred_element_type=jnp.float32)
            + br[...]
        )
        if wfc is None:
            out[...] = h
        else:
            out[...] = (
                jnp.dot(h, wr[...], preferred_element_type=jnp.float32) + bwr[...]
            )

    args = [s0, s1, c0, c1, feats, root, bias]
    if wfc is not None:
        args += [wfc, bfc]
    d_out = _OUT if wfc is not None else _H
    return pl.pallas_call(
        body,
        out_shape=jax.ShapeDtypeStruct((_N, d_out), jnp.float32),
    )(*args)


def kernel(x, edge_index, edge_attr, Wa1, ba1, Wb1, bb1, root1, bias1,
           Wa2, ba2, Wb2, bb2, root2, bias2, Wfc, bfc):
    pad = _EP - _E
    src2d = jnp.concatenate(
        [edge_index[0].astype(jnp.int32), jnp.zeros((pad,), jnp.int32)]
    ).reshape(_ROWS, _CHUNK)
    dst2d = jnp.concatenate(
        [edge_index[1].astype(jnp.int32), jnp.full((pad,), _N, jnp.int32)]
    ).reshape(_ROWS, _CHUNK)
    ea_pad = jnp.concatenate([edge_attr, jnp.zeros((pad, _DE), jnp.float32)])

    w2d1 = Wb1.reshape(_HID, _DIN, _H).transpose(1, 2, 0).reshape(_DIN, _H * _HID)
    bb1r = bb1.reshape(_DIN, _H)
    w2d2 = Wb2.reshape(_HID, _H, _H).transpose(1, 2, 0).reshape(_H, _H * _HID)
    bb2r = bb2.reshape(_H, _H)
    kmat = jnp.repeat(jnp.eye(_H, dtype=jnp.float32), _HID, axis=0)
    zeros_n = jnp.zeros((_NP, _H), jnp.float32)
    ones_c = jnp.ones((_CHUNK, _H), jnp.float32)

    xj = _gather_rows(x, src2d, _DIN)
    msg1 = _edge_messages(xj, ea_pad, Wa1, ba1.reshape(1, _HID), w2d1, bb1r, kmat)
    s1, cnt = _scatter_add(msg1, dst2d, zeros_n, ones_c, with_counts=True)
    h1 = _node_update(s1[0, :_N], s1[1, :_N], cnt[0, :_N], cnt[1, :_N],
                      x, root1, bias1.reshape(1, _H))

    h1j = _gather_rows(h1, src2d, _H)
    msg2 = _edge_messages(h1j, ea_pad, Wa2, ba2.reshape(1, _HID), w2d2, bb2r, kmat)
    (s2,) = _scatter_add(msg2, dst2d, zeros_n, ones_c, with_counts=False)
    out = _node_update(s2[0, :_N], s2[1, :_N], cnt[0, :_N], cnt[1, :_N],
                       h1, root2, bias2.reshape(1, _H), Wfc, bfc.reshape(1, _OUT))
    return out


# speedup vs baseline: 3.7802x; 1.5378x over previous
import functools

import jax
import jax.numpy as jnp
from jax import lax
from jax.experimental import pallas as pl
from jax.experimental.pallas import tpu as pltpu
from jax.experimental.pallas import tpu_sc as plsc

_N = 10000
_NP = 10112
_E = 160000
_EP = 163840
_DIN = 128
_DE = 16
_HID = 64
_H = 16
_OUT = 16

_CHUNK = 128
_ROWS = _EP // _CHUNK
_EB = 2048


def _sc_dims():
    try:
        info = plsc.get_sparse_core_info()
        return info.num_cores, info.num_subcores
    except Exception:
        return 2, 16


def _gather_rows(table, idx2d, d):
    nc, ns = _sc_dims()
    rpw = _ROWS // (nc * ns)
    n_tab = table.shape[0]
    tps = n_tab // ns
    mesh = plsc.VectorSubcoreMesh(core_axis_name="c", subcore_axis_name="s")

    @functools.partial(
        pl.kernel,
        out_type=jax.ShapeDtypeStruct((_EP, d), jnp.float32),
        mesh=mesh,
        scratch_types=[
            pltpu.VMEM((rpw, _CHUNK), jnp.int32),
            pltpu.VMEM((2, _CHUNK, d), jnp.float32),
            pltpu.VMEM_SHARED((n_tab, d), jnp.float32),
            pltpu.SemaphoreType.DMA,
            pltpu.SemaphoreType.DMA,
        ],
        compiler_params=pltpu.CompilerParams(use_tc_tiling_on_sc=(d >= 128)),
    )
    def k(table_hbm, idx_hbm, out_hbm, idx_v, rows_v, tab_sh, sem0, sem1):
        s = lax.axis_index("s")
        w = s * nc + lax.axis_index("c")
        t0 = s * tps
        pltpu.sync_copy(table_hbm.at[pl.ds(t0, tps)], tab_sh.at[pl.ds(t0, tps)])
        r0 = w * rpw
        pltpu.sync_copy(idx_hbm.at[pl.ds(r0, rpw)], idx_v)
        plsc.subcore_barrier()

        pltpu.async_copy(tab_sh.at[idx_v.at[0]], rows_v.at[0], sem0)

        def body(jj, carry):
            j0 = 2 * jj
            j1 = j0 + 1
            pltpu.async_copy(tab_sh.at[idx_v.at[j1]], rows_v.at[1], sem1)
            pltpu.make_async_copy(tab_sh.at[idx_v.at[j0]], rows_v.at[0], sem0).wait()
            pltpu.sync_copy(rows_v.at[0], out_hbm.at[pl.ds((r0 + j0) * _CHUNK, _CHUNK)])

            @pl.when(j0 + 2 < rpw)
            def _():
                pltpu.async_copy(tab_sh.at[idx_v.at[j0 + 2]], rows_v.at[0], sem0)

            pltpu.make_async_copy(tab_sh.at[idx_v.at[j1]], rows_v.at[1], sem1).wait()
            pltpu.sync_copy(rows_v.at[1], out_hbm.at[pl.ds((r0 + j1) * _CHUNK, _CHUNK)])
            return carry

        lax.fori_loop(0, rpw // 2, body, 0)

    return k(table, idx2d)


def _scatter_add(msg, dst2d, zeros_n, ones_c, with_counts):
    nc, ns = _sc_dims()
    rpw = _ROWS // (nc * ns)
    nz = _NP // ns
    mesh = plsc.VectorSubcoreMesh(core_axis_name="c", subcore_axis_name="s")

    out_types = [jax.ShapeDtypeStruct((nc, _NP, _H), jnp.float32)]
    scratch = [
        pltpu.VMEM((rpw, _CHUNK), jnp.int32),
        pltpu.VMEM((2, _CHUNK, _H), jnp.float32),
        pltpu.VMEM_SHARED((_NP, _H), jnp.float32),
        pltpu.SemaphoreType.DMA,
        pltpu.SemaphoreType.DMA,
    ]
    if with_counts:
        out_types.append(jax.ShapeDtypeStruct((nc, _NP, _H), jnp.float32))
        scratch += [
            pltpu.VMEM((_CHUNK, _H), jnp.float32),
            pltpu.VMEM_SHARED((_NP, _H), jnp.float32),
        ]

    @functools.partial(
        pl.kernel,
        out_type=tuple(out_types),
        mesh=mesh,
        scratch_types=scratch,
        compiler_params=pltpu.CompilerParams(use_tc_tiling_on_sc=False),
    )
    def k(msg_hbm, dst_hbm, zeros_hbm, ones_hbm, *rest):
        if with_counts:
            s_out, c_out, idx_v, mv, s_sh, sem0, sem1, ones_v, c_sh = rest
        else:
            s_out, idx_v, mv, s_sh, sem0, sem1 = rest
        c = lax.axis_index("c")
        s = lax.axis_index("s")
        w = s * nc + c
        rz = s * nz
        pltpu.sync_copy(zeros_hbm.at[pl.ds(rz, nz)], s_sh.at[pl.ds(rz, nz)])
        if with_counts:
            pltpu.sync_copy(zeros_hbm.at[pl.ds(rz, nz)], c_sh.at[pl.ds(rz, nz)])
            pltpu.sync_copy(ones_hbm, ones_v)
        r0 = w * rpw
        pltpu.sync_copy(dst_hbm.at[pl.ds(r0, rpw)], idx_v)
        plsc.subcore_barrier()

        pltpu.async_copy(msg_hbm.at[pl.ds(r0 * _CHUNK, _CHUNK)], mv.at[0], sem0)

        def body(jj, carry):
            j0 = 2 * jj
            j1 = j0 + 1
            pltpu.async_copy(
                msg_hbm.at[pl.ds((r0 + j1) * _CHUNK, _CHUNK)], mv.at[1], sem1)
            pltpu.make_async_copy(
                msg_hbm.at[pl.ds((r0 + j0) * _CHUNK, _CHUNK)], mv.at[0], sem0).wait()
            pltpu.sync_copy(mv.at[0], s_sh.at[idx_v.at[j0]], add=True)
            if with_counts:
                pltpu.sync_copy(ones_v, c_sh.at[idx_v.at[j0]], add=True)

            @pl.when(j0 + 2 < rpw)
            def _():
                pltpu.async_copy(
                    msg_hbm.at[pl.ds((r0 + j0 + 2) * _CHUNK, _CHUNK)], mv.at[0], sem0)

            pltpu.make_async_copy(
                msg_hbm.at[pl.ds((r0 + j1) * _CHUNK, _CHUNK)], mv.at[1], sem1).wait()
            pltpu.sync_copy(mv.at[1], s_sh.at[idx_v.at[j1]], add=True)
            if with_counts:
                pltpu.sync_copy(ones_v, c_sh.at[idx_v.at[j1]], add=True)
            return carry

        lax.fori_loop(0, rpw // 2, body, 0)

        plsc.subcore_barrier()
        pltpu.sync_copy(s_sh.at[pl.ds(rz, nz)], s_out.at[c, pl.ds(rz, nz)])
        if with_counts:
            pltpu.sync_copy(c_sh.at[pl.ds(rz, nz)], c_out.at[c, pl.ds(rz, nz)])

    return k(msg, dst2d, zeros_n, ones_c)


def _edge_messages1(xj, ea, Wa, ba, W2d_bf, bbr, kmat):
    grid = _EP // _EB

    def body(xj_ref, ea_ref, wa_ref, ba_ref, w2d_ref, bbr_ref, k_ref, out_ref):
        h = jax.nn.relu(
            jnp.dot(ea_ref[...], wa_ref[...], preferred_element_type=jnp.float32)
            + ba_ref[...]
        )
        xb = xj_ref[...].astype(jnp.bfloat16)
        g = jnp.dot(xb, w2d_ref[...], preferred_element_type=jnp.float32)
        hh = jnp.concatenate([h] * _H, axis=1)
        p = (g * hh).astype(jnp.bfloat16)
        out_ref[...] = (
            jnp.dot(p, k_ref[...], preferred_element_type=jnp.float32)
            + jnp.dot(xj_ref[...], bbr_ref[...], preferred_element_type=jnp.float32)
        )

    return pl.pallas_call(
        body,
        grid=(grid,),
        in_specs=[
            pl.BlockSpec((_EB, _DIN), lambda i: (i, 0)),
            pl.BlockSpec((_EB, _DE), lambda i: (i, 0)),
            pl.BlockSpec((_DE, _HID), lambda i: (0, 0)),
            pl.BlockSpec((1, _HID), lambda i: (0, 0)),
            pl.BlockSpec((_DIN, _H * _HID), lambda i: (0, 0)),
            pl.BlockSpec((_DIN, _H), lambda i: (0, 0)),
            pl.BlockSpec((_H * _HID, _H), lambda i: (0, 0)),
        ],
        out_specs=pl.BlockSpec((_EB, _H), lambda i: (i, 0)),
        out_shape=jax.ShapeDtypeStruct((_EP, _H), jnp.float32),
    )(xj, ea, Wa, ba, W2d_bf, bbr, kmat)


def _edge_messages2(h1j, ea, Wa, ba, Wb, bb, rmat, k16):
    grid = _EP // _EB

    def body(hj_ref, ea_ref, wa_ref, ba_ref, wb_ref, bb_ref, r_ref, k_ref, out_ref):
        h = jax.nn.relu(
            jnp.dot(ea_ref[...], wa_ref[...], preferred_element_type=jnp.float32)
            + ba_ref[...]
        )
        w2 = jnp.dot(h, wb_ref[...], preferred_element_type=jnp.float32) + bb_ref[...]
        rep = jnp.dot(hj_ref[...], r_ref[...], preferred_element_type=jnp.float32)
        out_ref[...] = jnp.dot(w2 * rep, k_ref[...], preferred_element_type=jnp.float32)

    return pl.pallas_call(
        body,
        grid=(grid,),
        in_specs=[
            pl.BlockSpec((_EB, _H), lambda i: (i, 0)),
            pl.BlockSpec((_EB, _DE), lambda i: (i, 0)),
            pl.BlockSpec((_DE, _HID), lambda i: (0, 0)),
            pl.BlockSpec((1, _HID), lambda i: (0, 0)),
            pl.BlockSpec((_HID, _H * _H), lambda i: (0, 0)),
            pl.BlockSpec((1, _H * _H), lambda i: (0, 0)),
            pl.BlockSpec((_H, _H * _H), lambda i: (0, 0)),
            pl.BlockSpec((_H * _H, _H), lambda i: (0, 0)),
        ],
        out_specs=pl.BlockSpec((_EB, _H), lambda i: (i, 0)),
        out_shape=jax.ShapeDtypeStruct((_EP, _H), jnp.float32),
    )(h1j, ea, Wa, ba, Wb, bb, rmat, k16)


def _node_update(s0, s1, c0, c1, feats, root, bias, wfc=None, bfc=None):
    def body(*refs):
        if wfc is None:
            s0r, s1r, c0r, c1r, fr, rr, br, out = refs
        else:
            s0r, s1r, c0r, c1r, fr, rr, br, wr, bwr, out = refs
        cnt = jnp.maximum(c0r[...] + c1r[...], 1.0)
        h = jax.nn.relu(
            (s0r[...] + s1r[...]) / cnt
            + jnp.dot(fr[...], rr[...], preferred_element_type=jnp.float32)
            + br[...]
        )
        if wfc is None:
            out[...] = h
        else:
            out[...] = (
                jnp.dot(h, wr[...], preferred_element_type=jnp.float32) + bwr[...]
            )

    args = [s0, s1, c0, c1, feats, root, bias]
    if wfc is not None:
        args += [wfc, bfc]
    d_out = _OUT if wfc is not None else _H
    return pl.pallas_call(
        body,
        out_shape=jax.ShapeDtypeStruct((_N, d_out), jnp.float32),
    )(*args)


def kernel(x, edge_index, edge_attr, Wa1, ba1, Wb1, bb1, root1, bias1,
           Wa2, ba2, Wb2, bb2, root2, bias2, Wfc, bfc):
    pad = _EP - _E
    src2d = jnp.concatenate(
        [edge_index[0].astype(jnp.int32), jnp.zeros((pad,), jnp.int32)]
    ).reshape(_ROWS, _CHUNK)
    dst2d = jnp.concatenate(
        [edge_index[1].astype(jnp.int32), jnp.full((pad,), _N, jnp.int32)]
    ).reshape(_ROWS, _CHUNK)
    ea_pad = jnp.concatenate([edge_attr, jnp.zeros((pad, _DE), jnp.float32)])

    w2d1 = (Wb1.reshape(_HID, _DIN, _H).transpose(1, 2, 0)
            .reshape(_DIN, _H * _HID).astype(jnp.bfloat16))
    bb1r = bb1.reshape(_DIN, _H)
    kmat = jnp.repeat(jnp.eye(_H, dtype=jnp.bfloat16), _HID, axis=0)
    rmat = jnp.repeat(jnp.eye(_H, dtype=jnp.float32), _H, axis=1)
    k16 = jnp.tile(jnp.eye(_H, dtype=jnp.float32), (_H, 1))
    zeros_n = jnp.zeros((_NP, _H), jnp.float32)
    ones_c = jnp.ones((_CHUNK, _H), jnp.float32)

    x_pad = jnp.concatenate([x, jnp.zeros((_NP - _N, _DIN), jnp.float32)])
    xj = _gather_rows(x_pad, src2d, _DIN)
    msg1 = _edge_messages1(xj, ea_pad, Wa1, ba1.reshape(1, _HID), w2d1, bb1r, kmat)
    s1, cnt = _scatter_add(msg1, dst2d, zeros_n, ones_c, with_counts=True)
    h1 = _node_update(s1[0, :_N], s1[1, :_N], cnt[0, :_N], cnt[1, :_N],
                      x, root1, bias1.reshape(1, _H))

    h1_pad = jnp.concatenate([h1, jnp.zeros((_NP - _N, _H), jnp.float32)])
    h1j = _gather_rows(h1_pad, src2d, _H)
    msg2 = _edge_messages2(h1j, ea_pad, Wa2, ba2.reshape(1, _HID), Wb2,
                           bb2.reshape(1, _H * _H), rmat, k16)
    (s2,) = _scatter_add(msg2, dst2d, zeros_n, ones_c, with_counts=False)
    out = _node_update(s2[0, :_N], s2[1, :_N], cnt[0, :_N], cnt[1, :_N],
                       h1, root2, bias2.reshape(1, _H), Wfc, bfc.reshape(1, _OUT))
    return out


# speedup vs baseline: 3.9640x; 1.0486x over previous
import functools

import jax
import jax.numpy as jnp
from jax import lax
from jax.experimental import pallas as pl
from jax.experimental.pallas import tpu as pltpu
from jax.experimental.pallas import tpu_sc as plsc

_N = 10000
_NP = 10112
_E = 160000
_EP = 163840
_DIN = 128
_DE = 16
_HID = 64
_H = 16
_OUT = 16

_CHUNK = 128
_ROWS = _EP // _CHUNK
_EB = 4096


def _sc_dims():
    try:
        info = plsc.get_sparse_core_info()
        return info.num_cores, info.num_subcores
    except Exception:
        return 2, 16


def _gather_rows(table, idx2d, d):
    nc, ns = _sc_dims()
    rpw = _ROWS // (nc * ns)
    n_tab = table.shape[0]
    tps = n_tab // ns
    mesh = plsc.VectorSubcoreMesh(core_axis_name="c", subcore_axis_name="s")

    if d <= 16:
        @functools.partial(
            pl.kernel,
            out_type=jax.ShapeDtypeStruct((_EP, d), jnp.float32),
            mesh=mesh,
            scratch_types=[
                pltpu.VMEM((rpw, _CHUNK), jnp.int32),
                pltpu.VMEM((rpw * _CHUNK, d), jnp.float32),
                pltpu.VMEM_SHARED((n_tab, d), jnp.float32),
                pltpu.SemaphoreType.DMA,
            ],
            compiler_params=pltpu.CompilerParams(use_tc_tiling_on_sc=False),
        )
        def kn(table_hbm, idx_hbm, out_hbm, idx_v, rows_v, tab_sh, sem):
            s = lax.axis_index("s")
            w = s * nc + lax.axis_index("c")
            t0 = s * tps
            pltpu.sync_copy(table_hbm.at[pl.ds(t0, tps)], tab_sh.at[pl.ds(t0, tps)])
            r0 = w * rpw
            pltpu.sync_copy(idx_hbm.at[pl.ds(r0, rpw)], idx_v)
            plsc.subcore_barrier()

            def fire(j, carry):
                pltpu.async_copy(
                    tab_sh.at[idx_v.at[j]],
                    rows_v.at[pl.ds(j * _CHUNK, _CHUNK)], sem)
                return carry

            lax.fori_loop(0, rpw, fire, 0)

            def drain(j, carry):
                pltpu.make_async_copy(
                    tab_sh.at[idx_v.at[j]],
                    rows_v.at[pl.ds(j * _CHUNK, _CHUNK)], sem).wait()
                return carry

            lax.fori_loop(0, rpw, drain, 0)
            pltpu.sync_copy(rows_v, out_hbm.at[pl.ds(r0 * _CHUNK, rpw * _CHUNK)])

        return kn(table, idx2d)

    @functools.partial(
        pl.kernel,
        out_type=jax.ShapeDtypeStruct((_EP, d), jnp.float32),
        mesh=mesh,
        scratch_types=[
            pltpu.VMEM((rpw, _CHUNK), jnp.int32),
            pltpu.VMEM((2, _CHUNK, d), jnp.float32),
            pltpu.VMEM_SHARED((n_tab, d), jnp.float32),
            pltpu.SemaphoreType.DMA,
            pltpu.SemaphoreType.DMA,
        ],
        compiler_params=pltpu.CompilerParams(use_tc_tiling_on_sc=(d >= 128)),
    )
    def k(table_hbm, idx_hbm, out_hbm, idx_v, rows_v, tab_sh, sem0, sem1):
        s = lax.axis_index("s")
        w = s * nc + lax.axis_index("c")
        t0 = s * tps
        pltpu.sync_copy(table_hbm.at[pl.ds(t0, tps)], tab_sh.at[pl.ds(t0, tps)])
        r0 = w * rpw
        pltpu.sync_copy(idx_hbm.at[pl.ds(r0, rpw)], idx_v)
        plsc.subcore_barrier()

        pltpu.async_copy(tab_sh.at[idx_v.at[0]], rows_v.at[0], sem0)

        def body(jj, carry):
            j0 = 2 * jj
            j1 = j0 + 1
            pltpu.async_copy(tab_sh.at[idx_v.at[j1]], rows_v.at[1], sem1)
            pltpu.make_async_copy(tab_sh.at[idx_v.at[j0]], rows_v.at[0], sem0).wait()
            pltpu.sync_copy(rows_v.at[0], out_hbm.at[pl.ds((r0 + j0) * _CHUNK, _CHUNK)])

            @pl.when(j0 + 2 < rpw)
            def _():
                pltpu.async_copy(tab_sh.at[idx_v.at[j0 + 2]], rows_v.at[0], sem0)

            pltpu.make_async_copy(tab_sh.at[idx_v.at[j1]], rows_v.at[1], sem1).wait()
            pltpu.sync_copy(rows_v.at[1], out_hbm.at[pl.ds((r0 + j1) * _CHUNK, _CHUNK)])
            return carry

        lax.fori_loop(0, rpw // 2, body, 0)

    return k(table, idx2d)


def _scatter_add(msg, dst2d, zeros_n, ones_c, with_counts):
    nc, ns = _sc_dims()
    rpw = _ROWS // (nc * ns)
    nz = _NP // ns
    mesh = plsc.VectorSubcoreMesh(core_axis_name="c", subcore_axis_name="s")

    out_types = [jax.ShapeDtypeStruct((nc, _NP, _H), jnp.float32)]
    scratch = [
        pltpu.VMEM((rpw, _CHUNK), jnp.int32),
        pltpu.VMEM((2, _CHUNK, _H), jnp.float32),
        pltpu.VMEM_SHARED((_NP, _H), jnp.float32),
        pltpu.SemaphoreType.DMA,
        pltpu.SemaphoreType.DMA,
    ]
    if with_counts:
        out_types.append(jax.ShapeDtypeStruct((nc, _NP, _H), jnp.float32))
        scratch += [
            pltpu.VMEM((_CHUNK, _H), jnp.float32),
            pltpu.VMEM_SHARED((_NP, _H), jnp.float32),
        ]

    @functools.partial(
        pl.kernel,
        out_type=tuple(out_types),
        mesh=mesh,
        scratch_types=scratch,
        compiler_params=pltpu.CompilerParams(use_tc_tiling_on_sc=False),
    )
    def k(msg_hbm, dst_hbm, zeros_hbm, ones_hbm, *rest):
        if with_counts:
            s_out, c_out, idx_v, mv, s_sh, sem0, sem1, ones_v, c_sh = rest
        else:
            s_out, idx_v, mv, s_sh, sem0, sem1 = rest
        c = lax.axis_index("c")
        s = lax.axis_index("s")
        w = s * nc + c
        rz = s * nz
        pltpu.sync_copy(zeros_hbm.at[pl.ds(rz, nz)], s_sh.at[pl.ds(rz, nz)])
        if with_counts:
            pltpu.sync_copy(zeros_hbm.at[pl.ds(rz, nz)], c_sh.at[pl.ds(rz, nz)])
            pltpu.sync_copy(ones_hbm, ones_v)
        r0 = w * rpw
        pltpu.sync_copy(dst_hbm.at[pl.ds(r0, rpw)], idx_v)
        plsc.subcore_barrier()

        pltpu.async_copy(msg_hbm.at[pl.ds(r0 * _CHUNK, _CHUNK)], mv.at[0], sem0)

        def body(jj, carry):
            j0 = 2 * jj
            j1 = j0 + 1
            pltpu.async_copy(
                msg_hbm.at[pl.ds((r0 + j1) * _CHUNK, _CHUNK)], mv.at[1], sem1)
            pltpu.make_async_copy(
                msg_hbm.at[pl.ds((r0 + j0) * _CHUNK, _CHUNK)], mv.at[0], sem0).wait()
            pltpu.sync_copy(mv.at[0], s_sh.at[idx_v.at[j0]], add=True)
            if with_counts:
                pltpu.sync_copy(ones_v, c_sh.at[idx_v.at[j0]], add=True)

            @pl.when(j0 + 2 < rpw)
            def _():
                pltpu.async_copy(
                    msg_hbm.at[pl.ds((r0 + j0 + 2) * _CHUNK, _CHUNK)], mv.at[0], sem0)

            pltpu.make_async_copy(
                msg_hbm.at[pl.ds((r0 + j1) * _CHUNK, _CHUNK)], mv.at[1], sem1).wait()
            pltpu.sync_copy(mv.at[1], s_sh.at[idx_v.at[j1]], add=True)
            if with_counts:
                pltpu.sync_copy(ones_v, c_sh.at[idx_v.at[j1]], add=True)
            return carry

        lax.fori_loop(0, rpw // 2, body, 0)

        plsc.subcore_barrier()
        pltpu.sync_copy(s_sh.at[pl.ds(rz, nz)], s_out.at[c, pl.ds(rz, nz)])
        if with_counts:
            pltpu.sync_copy(c_sh.at[pl.ds(rz, nz)], c_out.at[c, pl.ds(rz, nz)])

    return k(msg, dst2d, zeros_n, ones_c)


def _edge_messages1(xj, ea, Wa, ba, W2d_bf, bbr, kmat):
    grid = _EP // _EB

    def body(xj_ref, ea_ref, wa_ref, ba_ref, w2d_ref, bbr_ref, k_ref, out_ref):
        h = jax.nn.relu(
            jnp.dot(ea_ref[...].astype(jnp.bfloat16), wa_ref[...],
                    preferred_element_type=jnp.float32)
            + ba_ref[...]
        ).astype(jnp.bfloat16)
        xb = xj_ref[...].astype(jnp.bfloat16)
        g = jnp.dot(xb, w2d_ref[...],
                    preferred_element_type=jnp.float32).astype(jnp.bfloat16)
        hh = jnp.concatenate([h] * _H, axis=1)
        p = g * hh
        out_ref[...] = (
            jnp.dot(p, k_ref[...], preferred_element_type=jnp.float32)
            + jnp.dot(xb, bbr_ref[...], preferred_element_type=jnp.float32)
        )

    return pl.pallas_call(
        body,
        grid=(grid,),
        in_specs=[
            pl.BlockSpec((_EB, _DIN), lambda i: (i, 0)),
            pl.BlockSpec((_EB, _DE), lambda i: (i, 0)),
            pl.BlockSpec((_DE, _HID), lambda i: (0, 0)),
            pl.BlockSpec((1, _HID), lambda i: (0, 0)),
            pl.BlockSpec((_DIN, _H * _HID), lambda i: (0, 0)),
            pl.BlockSpec((_DIN, _H), lambda i: (0, 0)),
            pl.BlockSpec((_H * _HID, _H), lambda i: (0, 0)),
        ],
        out_specs=pl.BlockSpec((_EB, _H), lambda i: (i, 0)),
        out_shape=jax.ShapeDtypeStruct((_EP, _H), jnp.float32),
    )(xj, ea, Wa, ba, W2d_bf, bbr, kmat)


def _edge_messages2(h1j, ea, Wa, ba, Wb, bb, rmat, k16):
    grid = _EP // _EB

    def body(hj_ref, ea_ref, wa_ref, ba_ref, wb_ref, bb_ref, r_ref, k_ref, out_ref):
        h = jax.nn.relu(
            jnp.dot(ea_ref[...], wa_ref[...], preferred_element_type=jnp.float32)
            + ba_ref[...]
        )
        w2 = jnp.dot(h, wb_ref[...], preferred_element_type=jnp.float32) + bb_ref[...]
        rep = jnp.dot(hj_ref[...].astype(jnp.bfloat16), r_ref[...],
                      preferred_element_type=jnp.float32)
        p = (w2 * rep).astype(jnp.bfloat16)
        out_ref[...] = jnp.dot(p, k_ref[...], preferred_element_type=jnp.float32)

    return pl.pallas_call(
        body,
        grid=(grid,),
        in_specs=[
            pl.BlockSpec((_EB, _H), lambda i: (i, 0)),
            pl.BlockSpec((_EB, _DE), lambda i: (i, 0)),
            pl.BlockSpec((_DE, _HID), lambda i: (0, 0)),
            pl.BlockSpec((1, _HID), lambda i: (0, 0)),
            pl.BlockSpec((_HID, _H * _H), lambda i: (0, 0)),
            pl.BlockSpec((1, _H * _H), lambda i: (0, 0)),
            pl.BlockSpec((_H, _H * _H), lambda i: (0, 0)),
            pl.BlockSpec((_H * _H, _H), lambda i: (0, 0)),
        ],
        out_specs=pl.BlockSpec((_EB, _H), lambda i: (i, 0)),
        out_shape=jax.ShapeDtypeStruct((_EP, _H), jnp.float32),
    )(h1j, ea, Wa, ba, Wb, bb, rmat, k16)


def _node_update(s0, s1, c0, c1, feats, root, bias, wfc=None, bfc=None):
    def body(*refs):
        if wfc is None:
            s0r, s1r, c0r, c1r, fr, rr, br, out = refs
        else:
            s0r, s1r, c0r, c1r, fr, rr, br, wr, bwr, out = refs
        cnt = jnp.maximum(c0r[...] + c1r[...], 1.0)
        h = jax.nn.relu(
            (s0r[...] + s1r[...]) / cnt
            + jnp.dot(fr[...], rr[...], preferred_element_type=jnp.float32)
            + br[...]
        )
        if wfc is None:
            out[...] = h
        else:
            out[...] = (
                jnp.dot(h, wr[...], preferred_element_type=jnp.float32) + bwr[...]
            )

    args = [s0, s1, c0, c1, feats, root, bias]
    if wfc is not None:
        args += [wfc, bfc]
    d_out = _OUT if wfc is not None else _H
    return pl.pallas_call(
        body,
        out_shape=jax.ShapeDtypeStruct((_N, d_out), jnp.float32),
    )(*args)


def kernel(x, edge_index, edge_attr, Wa1, ba1, Wb1, bb1, root1, bias1,
           Wa2, ba2, Wb2, bb2, root2, bias2, Wfc, bfc):
    pad = _EP - _E
    src2d = jnp.concatenate(
        [edge_index[0].astype(jnp.int32), jnp.zeros((pad,), jnp.int32)]
    ).reshape(_ROWS, _CHUNK)
    dst2d = jnp.concatenate(
        [edge_index[1].astype(jnp.int32), jnp.full((pad,), _N, jnp.int32)]
    ).reshape(_ROWS, _CHUNK)
    ea_pad = jnp.concatenate([edge_attr, jnp.zeros((pad, _DE), jnp.float32)])

    w2d1 = (Wb1.reshape(_HID, _DIN, _H).transpose(1, 2, 0)
            .reshape(_DIN, _H * _HID).astype(jnp.bfloat16))
    bb1r = bb1.reshape(_DIN, _H).astype(jnp.bfloat16)
    wa1_bf = Wa1.astype(jnp.bfloat16)
    kmat = jnp.repeat(jnp.eye(_H, dtype=jnp.bfloat16), _HID, axis=0)
    rmat = jnp.repeat(jnp.eye(_H, dtype=jnp.bfloat16), _H, axis=1)
    k16 = jnp.tile(jnp.eye(_H, dtype=jnp.bfloat16), (_H, 1))
    zeros_n = jnp.zeros((_NP, _H), jnp.float32)
    ones_c = jnp.ones((_CHUNK, _H), jnp.float32)

    x_pad = jnp.concatenate([x, jnp.zeros((_NP - _N, _DIN), jnp.float32)])
    xj = _gather_rows(x_pad, src2d, _DIN)
    msg1 = _edge_messages1(xj, ea_pad, wa1_bf, ba1.reshape(1, _HID), w2d1, bb1r, kmat)
    s1, cnt = _scatter_add(msg1, dst2d, zeros_n, ones_c, with_counts=True)
    h1 = _node_update(s1[0, :_N], s1[1, :_N], cnt[0, :_N], cnt[1, :_N],
                      x, root1, bias1.reshape(1, _H))

    h1_pad = jnp.concatenate([h1, jnp.zeros((_NP - _N, _H), jnp.float32)])
    h1j = _gather_rows(h1_pad, src2d, _H)
    msg2 = _edge_messages2(h1j, ea_pad, Wa2, ba2.reshape(1, _HID), Wb2,
                           bb2.reshape(1, _H * _H), rmat, k16)
    (s2,) = _scatter_add(msg2, dst2d, zeros_n, ones_c, with_counts=False)
    out = _node_update(s2[0, :_N], s2[1, :_N], cnt[0, :_N], cnt[1, :_N],
                       h1, root2, bias2.reshape(1, _H), Wfc, bfc.reshape(1, _OUT))
    return out


# speedup vs baseline: 4.7177x; 1.1901x over previous
import functools

import jax
import jax.numpy as jnp
from jax import lax
from jax.experimental import pallas as pl
from jax.experimental.pallas import tpu as pltpu
from jax.experimental.pallas import tpu_sc as plsc

_N = 10000
_NP = 10112
_E = 160000
_EP = 163840
_DIN = 128
_DE = 16
_HID = 64
_H = 16
_OUT = 16

_CHUNK = 128
_ROWS = _EP // _CHUNK
_EB = 4000
_W = 128


def _sc_dims():
    try:
        info = plsc.get_sparse_core_info()
        return info.num_cores, info.num_subcores
    except Exception:
        return 2, 16


def _gather_rows(table, idx2d, d):
    nc, ns = _sc_dims()
    rpw = _ROWS // (nc * ns)
    n_tab = table.shape[0]
    tps = (n_tab // ns) // 8 * 8
    tail = n_tab - ns * tps
    mesh = plsc.VectorSubcoreMesh(core_axis_name="c", subcore_axis_name="s")

    if d <= 16:
        @functools.partial(
            pl.kernel,
            out_type=jax.ShapeDtypeStruct((_EP, d), jnp.float32),
            mesh=mesh,
            scratch_types=[
                pltpu.VMEM((rpw, _CHUNK), jnp.int32),
                pltpu.VMEM((rpw * _CHUNK, d), jnp.float32),
                pltpu.VMEM_SHARED((n_tab, d), jnp.float32),
                pltpu.SemaphoreType.DMA,
            ],
            compiler_params=pltpu.CompilerParams(use_tc_tiling_on_sc=False),
        )
        def kn(table_hbm, idx_hbm, out_hbm, idx_v, rows_v, tab_sh, sem):
            s = lax.axis_index("s")
            w = s * nc + lax.axis_index("c")
            t0 = s * tps
            pltpu.sync_copy(table_hbm.at[pl.ds(t0, tps)], tab_sh.at[pl.ds(t0, tps)])
            r0 = w * rpw
            pltpu.sync_copy(idx_hbm.at[pl.ds(r0, rpw)], idx_v)
            plsc.subcore_barrier()

            def fire(j, carry):
                pltpu.async_copy(
                    tab_sh.at[idx_v.at[j]],
                    rows_v.at[pl.ds(j * _CHUNK, _CHUNK)], sem)
                return carry

            lax.fori_loop(0, rpw, fire, 0)

            def drain(j, carry):
                pltpu.make_async_copy(
                    tab_sh.at[idx_v.at[j]],
                    rows_v.at[pl.ds(j * _CHUNK, _CHUNK)], sem).wait()
                return carry

            lax.fori_loop(0, rpw, drain, 0)
            pltpu.sync_copy(rows_v, out_hbm.at[pl.ds(r0 * _CHUNK, rpw * _CHUNK)])

        return kn(table, idx2d)

    @functools.partial(
        pl.kernel,
        out_type=jax.ShapeDtypeStruct((_EP, d), jnp.float32),
        mesh=mesh,
        scratch_types=[
            pltpu.VMEM((rpw, _CHUNK), jnp.int32),
            pltpu.VMEM((2, _CHUNK, d), jnp.float32),
            pltpu.VMEM_SHARED((n_tab, d), jnp.float32),
            pltpu.SemaphoreType.DMA,
            pltpu.SemaphoreType.DMA,
        ],
        compiler_params=pltpu.CompilerParams(use_tc_tiling_on_sc=(d >= 128)),
    )
    def k(table_hbm, idx_hbm, out_hbm, idx_v, rows_v, tab_sh, sem0, sem1):
        s = lax.axis_index("s")
        w = s * nc + lax.axis_index("c")
        t0 = s * tps
        pltpu.sync_copy(table_hbm.at[pl.ds(t0, tps)], tab_sh.at[pl.ds(t0, tps)])
        if tail:
            @pl.when(s == ns - 1)
            def _():
                pltpu.sync_copy(table_hbm.at[pl.ds(ns * tps, tail)],
                                tab_sh.at[pl.ds(ns * tps, tail)])
        r0 = w * rpw
        pltpu.sync_copy(idx_hbm.at[pl.ds(r0, rpw)], idx_v)
        plsc.subcore_barrier()

        pltpu.async_copy(tab_sh.at[idx_v.at[0]], rows_v.at[0], sem0)

        def body(jj, carry):
            j0 = 2 * jj
            j1 = j0 + 1
            pltpu.async_copy(tab_sh.at[idx_v.at[j1]], rows_v.at[1], sem1)
            pltpu.make_async_copy(tab_sh.at[idx_v.at[j0]], rows_v.at[0], sem0).wait()
            pltpu.sync_copy(rows_v.at[0], out_hbm.at[pl.ds((r0 + j0) * _CHUNK, _CHUNK)])

            @pl.when(j0 + 2 < rpw)
            def _():
                pltpu.async_copy(tab_sh.at[idx_v.at[j0 + 2]], rows_v.at[0], sem0)

            pltpu.make_async_copy(tab_sh.at[idx_v.at[j1]], rows_v.at[1], sem1).wait()
            pltpu.sync_copy(rows_v.at[1], out_hbm.at[pl.ds((r0 + j1) * _CHUNK, _CHUNK)])
            return carry

        lax.fori_loop(0, rpw // 2, body, 0)

    return k(table, idx2d)


def _scatter_add(msg, dst2d, zeros_n):
    nc, ns = _sc_dims()
    rpw = _ROWS // (nc * ns)
    nz = _NP // ns
    mesh = plsc.VectorSubcoreMesh(core_axis_name="c", subcore_axis_name="s")

    @functools.partial(
        pl.kernel,
        out_type=jax.ShapeDtypeStruct((nc, _NP, _W), jnp.float32),
        mesh=mesh,
        scratch_types=[
            pltpu.VMEM((rpw, _CHUNK), jnp.int32),
            pltpu.VMEM((2, _CHUNK, _W), jnp.float32),
            pltpu.VMEM_SHARED((_NP, _W), jnp.float32),
            pltpu.SemaphoreType.DMA,
            pltpu.SemaphoreType.DMA,
        ],
    )
    def k(msg_hbm, dst_hbm, zeros_hbm, s_out, idx_v, mv, s_sh, sem0, sem1):
        c = lax.axis_index("c")
        s = lax.axis_index("s")
        w = s * nc + c
        rz = s * nz
        pltpu.sync_copy(zeros_hbm.at[pl.ds(rz, nz)], s_sh.at[pl.ds(rz, nz)])
        r0 = w * rpw
        pltpu.sync_copy(dst_hbm.at[pl.ds(r0, rpw)], idx_v)
        plsc.subcore_barrier()

        pltpu.async_copy(msg_hbm.at[pl.ds(r0 * _CHUNK, _CHUNK)], mv.at[0], sem0)

        def body(jj, carry):
            j0 = 2 * jj
            j1 = j0 + 1
            pltpu.async_copy(
                msg_hbm.at[pl.ds((r0 + j1) * _CHUNK, _CHUNK)], mv.at[1], sem1)
            pltpu.make_async_copy(
                msg_hbm.at[pl.ds((r0 + j0) * _CHUNK, _CHUNK)], mv.at[0], sem0).wait()
            pltpu.sync_copy(mv.at[0], s_sh.at[idx_v.at[j0]], add=True)

            @pl.when(j0 + 2 < rpw)
            def _():
                pltpu.async_copy(
                    msg_hbm.at[pl.ds((r0 + j0 + 2) * _CHUNK, _CHUNK)], mv.at[0], sem0)

            pltpu.make_async_copy(
                msg_hbm.at[pl.ds((r0 + j1) * _CHUNK, _CHUNK)], mv.at[1], sem1).wait()
            pltpu.sync_copy(mv.at[1], s_sh.at[idx_v.at[j1]], add=True)
            return carry

        lax.fori_loop(0, rpw // 2, body, 0)

        plsc.subcore_barrier()
        pltpu.sync_copy(s_sh.at[pl.ds(rz, nz)], s_out.at[c, pl.ds(rz, nz)])

    return k(msg, dst2d, zeros_n)


def _edge_messages1(xj, ea_bf, Wa, ba, W2d_bf, bbr, kmat):
    grid = _E // _EB

    def body(xj_ref, ea_ref, wa_ref, ba_ref, w2d_ref, bbr_ref, k_ref, out_ref):
        h = jax.nn.relu(
            jnp.dot(ea_ref[...], wa_ref[...], preferred_element_type=jnp.float32)
            + ba_ref[...]
        ).astype(jnp.bfloat16)
        xb = xj_ref[...].astype(jnp.bfloat16)
        g = jnp.dot(xb, w2d_ref[...],
                    preferred_element_type=jnp.float32).astype(jnp.bfloat16)
        hh = jnp.concatenate([h] * _H, axis=1)
        p = g * hh
        out_ref[:, 0:_H] = (
            jnp.dot(p, k_ref[...], preferred_element_type=jnp.float32)
            + jnp.dot(xb, bbr_ref[...], preferred_element_type=jnp.float32)
        )
        out_ref[:, _H:_H + 1] = jnp.ones((_EB, 1), jnp.float32)

    return pl.pallas_call(
        body,
        grid=(grid,),
        in_specs=[
            pl.BlockSpec((_EB, _DIN), lambda i: (i, 0)),
            pl.BlockSpec((_EB, _DE), lambda i: (i, 0)),
            pl.BlockSpec((_DE, _HID), lambda i: (0, 0)),
            pl.BlockSpec((1, _HID), lambda i: (0, 0)),
            pl.BlockSpec((_DIN, _H * _HID), lambda i: (0, 0)),
            pl.BlockSpec((_DIN, _H), lambda i: (0, 0)),
            pl.BlockSpec((_H * _HID, _H), lambda i: (0, 0)),
        ],
        out_specs=pl.BlockSpec((_EB, _W), lambda i: (i, 0)),
        out_shape=jax.ShapeDtypeStruct((_EP, _W), jnp.float32),
    )(xj, ea_bf, Wa, ba, W2d_bf, bbr, kmat)


def _edge_messages2(h1j, ea_bf, Wa, ba, Wb, bb, rmat, k16):
    grid = _E // _EB

    def body(hj_ref, ea_ref, wa_ref, ba_ref, wb_ref, bb_ref, r_ref, k_ref, out_ref):
        h = jax.nn.relu(
            jnp.dot(ea_ref[...], wa_ref[...], preferred_element_type=jnp.float32)
            + ba_ref[...]
        )
        w2 = jnp.dot(h, wb_ref[...], preferred_element_type=jnp.float32) + bb_ref[...]
        rep = jnp.dot(hj_ref[...].astype(jnp.bfloat16), r_ref[...],
                      preferred_element_type=jnp.float32)
        p = (w2 * rep).astype(jnp.bfloat16)
        out_ref[:, 0:_H] = jnp.dot(p, k_ref[...], preferred_element_type=jnp.float32)
        out_ref[:, _H:_H + 1] = jnp.ones((_EB, 1), jnp.float32)

    return pl.pallas_call(
        body,
        grid=(grid,),
        in_specs=[
            pl.BlockSpec((_EB, _H), lambda i: (i, 0)),
            pl.BlockSpec((_EB, _DE), lambda i: (i, 0)),
            pl.BlockSpec((_DE, _HID), lambda i: (0, 0)),
            pl.BlockSpec((1, _HID), lambda i: (0, 0)),
            pl.BlockSpec((_HID, _H * _H), lambda i: (0, 0)),
            pl.BlockSpec((1, _H * _H), lambda i: (0, 0)),
            pl.BlockSpec((_H, _H * _H), lambda i: (0, 0)),
            pl.BlockSpec((_H * _H, _H), lambda i: (0, 0)),
        ],
        out_specs=pl.BlockSpec((_EB, _W), lambda i: (i, 0)),
        out_shape=jax.ShapeDtypeStruct((_EP, _W), jnp.float32),
    )(h1j, ea_bf, Wa, ba, Wb, bb, rmat, k16)


def _node_update(s0, s1, feats, root, bias, wfc=None, bfc=None):
    def body(*refs):
        if wfc is None:
            s0r, s1r, fr, rr, br, out = refs
        else:
            s0r, s1r, fr, rr, br, wr, bwr, out = refs
        ssum = s0r[0:_N, :] + s1r[0:_N, :]
        cnt = jnp.maximum(ssum[:, _H:_H + 1], 1.0)
        h = jax.nn.relu(
            ssum[:, 0:_H] / cnt
            + jnp.dot(fr[...], rr[...], preferred_element_type=jnp.float32)
            + br[...]
        )
        if wfc is None:
            out[...] = h
        else:
            out[...] = (
                jnp.dot(h, wr[...], preferred_element_type=jnp.float32) + bwr[...]
            )

    args = [s0, s1, feats, root, bias]
    if wfc is not None:
        args += [wfc, bfc]
    d_out = _OUT if wfc is not None else _H
    return pl.pallas_call(
        body,
        out_shape=jax.ShapeDtypeStruct((_N, d_out), jnp.float32),
    )(*args)


def kernel(x, edge_index, edge_attr, Wa1, ba1, Wb1, bb1, root1, bias1,
           Wa2, ba2, Wb2, bb2, root2, bias2, Wfc, bfc):
    pad = _EP - _E
    src2d = jnp.concatenate(
        [edge_index[0].astype(jnp.int32), jnp.zeros((pad,), jnp.int32)]
    ).reshape(_ROWS, _CHUNK)
    dst2d = jnp.concatenate(
        [edge_index[1].astype(jnp.int32), jnp.full((pad,), _N, jnp.int32)]
    ).reshape(_ROWS, _CHUNK)
    ea_bf = edge_attr.astype(jnp.bfloat16)

    w2d1 = (Wb1.reshape(_HID, _DIN, _H).transpose(1, 2, 0)
            .reshape(_DIN, _H * _HID).astype(jnp.bfloat16))
    bb1r = bb1.reshape(_DIN, _H).astype(jnp.bfloat16)
    wa1_bf = Wa1.astype(jnp.bfloat16)
    wa2_bf = Wa2.astype(jnp.bfloat16)
    kmat = jnp.repeat(jnp.eye(_H, dtype=jnp.bfloat16), _HID, axis=0)
    rmat = jnp.repeat(jnp.eye(_H, dtype=jnp.bfloat16), _H, axis=1)
    k16 = jnp.tile(jnp.eye(_H, dtype=jnp.bfloat16), (_H, 1))
    zeros_n = jnp.zeros((_NP, _W), jnp.float32)

    xj = _gather_rows(x, src2d, _DIN)
    msg1 = _edge_messages1(xj, ea_bf, wa1_bf, ba1.reshape(1, _HID), w2d1, bb1r, kmat)
    s1 = _scatter_add(msg1, dst2d, zeros_n)
    h1 = _node_update(s1[0], s1[1], x, root1, bias1.reshape(1, _H))

    h1_pad = jnp.concatenate([h1, jnp.zeros((_NP - _N, _H), jnp.float32)])
    h1j = _gather_rows(h1_pad, src2d, _H)
    msg2 = _edge_messages2(h1j, ea_bf, wa2_bf, ba2.reshape(1, _HID), Wb2,
                           bb2.reshape(1, _H * _H), rmat, k16)
    s2 = _scatter_add(msg2, dst2d, zeros_n)
    out = _node_update(s2[0], s2[1], h1, root2, bias2.reshape(1, _H),
                       Wfc, bfc.reshape(1, _OUT))
    return out


# speedup vs baseline: 5.7089x; 1.2101x over previous
import functools

import jax
import jax.numpy as jnp
from jax import lax
from jax.experimental import pallas as pl
from jax.experimental.pallas import tpu as pltpu
from jax.experimental.pallas import tpu_sc as plsc

_N = 10000
_NP = 10112
_E = 160000
_EP = 163840
_DIN = 128
_DE = 16
_HID = 64
_H = 16
_OUT = 16

_CHUNK = 128
_ROWS = _EP // _CHUNK
_EB = 4000
_W = 128
_SW = 32


def _sc_dims():
    try:
        info = plsc.get_sparse_core_info()
        return info.num_cores, info.num_subcores
    except Exception:
        return 2, 16


def _gather_rows(table, idx2d, d):
    nc, ns = _sc_dims()
    rpw = _ROWS // (nc * ns)
    n_tab = table.shape[0]
    tps = (n_tab // ns) // 8 * 8
    tail = n_tab - ns * tps
    mesh = plsc.VectorSubcoreMesh(core_axis_name="c", subcore_axis_name="s")

    if d <= 16:
        @functools.partial(
            pl.kernel,
            out_type=jax.ShapeDtypeStruct((_EP, _W), jnp.float32),
            mesh=mesh,
            scratch_types=[
                pltpu.VMEM((rpw, _CHUNK), jnp.int32),
                pltpu.VMEM((rpw * _CHUNK, d), jnp.float32),
                pltpu.VMEM_SHARED((n_tab, d), jnp.float32),
                pltpu.SemaphoreType.DMA,
            ],
            compiler_params=pltpu.CompilerParams(use_tc_tiling_on_sc=False),
        )
        def kn(table_hbm, idx_hbm, out_hbm, idx_v, rows_v, tab_sh, sem):
            s = lax.axis_index("s")
            w = s * nc + lax.axis_index("c")
            t0 = s * tps
            pltpu.sync_copy(table_hbm.at[pl.ds(t0, tps), pl.ds(0, d)],
                            tab_sh.at[pl.ds(t0, tps)])
            if tail:
                @pl.when(s == ns - 1)
                def _():
                    pltpu.sync_copy(table_hbm.at[pl.ds(ns * tps, tail), pl.ds(0, d)],
                                    tab_sh.at[pl.ds(ns * tps, tail)])
            r0 = w * rpw
            pltpu.sync_copy(idx_hbm.at[pl.ds(r0, rpw)], idx_v)
            plsc.subcore_barrier()

            def fire(j, carry):
                pltpu.async_copy(
                    tab_sh.at[idx_v.at[j]],
                    rows_v.at[pl.ds(j * _CHUNK, _CHUNK)], sem)
                return carry

            lax.fori_loop(0, rpw, fire, 0)

            def drain(j, carry):
                pltpu.make_async_copy(
                    tab_sh.at[idx_v.at[j]],
                    rows_v.at[pl.ds(j * _CHUNK, _CHUNK)], sem).wait()
                return carry

            lax.fori_loop(0, rpw, drain, 0)
            pltpu.sync_copy(rows_v,
                            out_hbm.at[pl.ds(r0 * _CHUNK, rpw * _CHUNK), pl.ds(0, d)])

        return kn(table, idx2d)

    @functools.partial(
        pl.kernel,
        out_type=jax.ShapeDtypeStruct((_EP, d), jnp.float32),
        mesh=mesh,
        scratch_types=[
            pltpu.VMEM((rpw, _CHUNK), jnp.int32),
            pltpu.VMEM((2, _CHUNK, d), jnp.float32),
            pltpu.VMEM_SHARED((n_tab, d), jnp.float32),
            pltpu.SemaphoreType.DMA,
            pltpu.SemaphoreType.DMA,
        ],
        compiler_params=pltpu.CompilerParams(use_tc_tiling_on_sc=(d >= 128)),
    )
    def k(table_hbm, idx_hbm, out_hbm, idx_v, rows_v, tab_sh, sem0, sem1):
        s = lax.axis_index("s")
        w = s * nc + lax.axis_index("c")
        t0 = s * tps
        pltpu.sync_copy(table_hbm.at[pl.ds(t0, tps)], tab_sh.at[pl.ds(t0, tps)])
        if tail:
            @pl.when(s == ns - 1)
            def _():
                pltpu.sync_copy(table_hbm.at[pl.ds(ns * tps, tail)],
                                tab_sh.at[pl.ds(ns * tps, tail)])
        r0 = w * rpw
        pltpu.sync_copy(idx_hbm.at[pl.ds(r0, rpw)], idx_v)
        plsc.subcore_barrier()

        pltpu.async_copy(tab_sh.at[idx_v.at[0]], rows_v.at[0], sem0)

        def body(jj, carry):
            j0 = 2 * jj
            j1 = j0 + 1
            pltpu.async_copy(tab_sh.at[idx_v.at[j1]], rows_v.at[1], sem1)
            pltpu.make_async_copy(tab_sh.at[idx_v.at[j0]], rows_v.at[0], sem0).wait()
            pltpu.sync_copy(rows_v.at[0], out_hbm.at[pl.ds((r0 + j0) * _CHUNK, _CHUNK)])

            @pl.when(j0 + 2 < rpw)
            def _():
                pltpu.async_copy(tab_sh.at[idx_v.at[j0 + 2]], rows_v.at[0], sem0)

            pltpu.make_async_copy(tab_sh.at[idx_v.at[j1]], rows_v.at[1], sem1).wait()
            pltpu.sync_copy(rows_v.at[1], out_hbm.at[pl.ds((r0 + j1) * _CHUNK, _CHUNK)])
            return carry

        lax.fori_loop(0, rpw // 2, body, 0)

    return k(table, idx2d)


def _scatter_add(msg, dst2d, zeros_n):
    nc, ns = _sc_dims()
    rpw = _ROWS // (nc * ns)
    nz = _NP // ns
    mesh = plsc.VectorSubcoreMesh(core_axis_name="c", subcore_axis_name="s")

    @functools.partial(
        pl.kernel,
        out_type=jax.ShapeDtypeStruct((nc, _NP, _W), jnp.float32),
        mesh=mesh,
        scratch_types=[
            pltpu.VMEM((rpw, _CHUNK), jnp.int32),
            pltpu.VMEM((2, _CHUNK, _SW), jnp.float32),
            pltpu.VMEM_SHARED((_NP, _SW), jnp.float32),
            pltpu.SemaphoreType.DMA,
            pltpu.SemaphoreType.DMA,
        ],
        compiler_params=pltpu.CompilerParams(use_tc_tiling_on_sc=False),
    )
    def k(msg_hbm, dst_hbm, zeros_hbm, s_out, idx_v, mv, s_sh, sem0, sem1):
        c = lax.axis_index("c")
        s = lax.axis_index("s")
        w = s * nc + c
        rz = s * nz
        pltpu.sync_copy(zeros_hbm.at[pl.ds(rz, nz), pl.ds(0, _SW)],
                        s_sh.at[pl.ds(rz, nz)])
        r0 = w * rpw
        pltpu.sync_copy(dst_hbm.at[pl.ds(r0, rpw)], idx_v)
        plsc.subcore_barrier()

        def load(j, buf, sem):
            return pltpu.async_copy(
                msg_hbm.at[pl.ds((r0 + j) * _CHUNK, _CHUNK), pl.ds(0, _SW)],
                buf, sem)

        load(0, mv.at[0], sem0)

        def body(jj, carry):
            j0 = 2 * jj
            j1 = j0 + 1
            load(j1, mv.at[1], sem1)
            pltpu.make_async_copy(
                msg_hbm.at[pl.ds((r0 + j0) * _CHUNK, _CHUNK), pl.ds(0, _SW)],
                mv.at[0], sem0).wait()
            pltpu.sync_copy(mv.at[0], s_sh.at[idx_v.at[j0]], add=True)

            @pl.when(j0 + 2 < rpw)
            def _():
                load(j0 + 2, mv.at[0], sem0)

            pltpu.make_async_copy(
                msg_hbm.at[pl.ds((r0 + j1) * _CHUNK, _CHUNK), pl.ds(0, _SW)],
                mv.at[1], sem1).wait()
            pltpu.sync_copy(mv.at[1], s_sh.at[idx_v.at[j1]], add=True)
            return carry

        lax.fori_loop(0, rpw // 2, body, 0)

        plsc.subcore_barrier()
        pltpu.sync_copy(s_sh.at[pl.ds(rz, nz)],
                        s_out.at[c, pl.ds(rz, nz), pl.ds(0, _SW)])

    return k(msg, dst2d, zeros_n)


def _edge_messages1(xj, ea_bf, Wa, ba, W2d_bf, bbr, kmat):
    grid = _E // _EB

    def body(xj_ref, ea_ref, wa_ref, ba_ref, w2d_ref, bbr_ref, k_ref, out_ref):
        h = jax.nn.relu(
            jnp.dot(ea_ref[...], wa_ref[...], preferred_element_type=jnp.float32)
            + ba_ref[...]
        ).astype(jnp.bfloat16)
        xb = xj_ref[...].astype(jnp.bfloat16)
        g = jnp.dot(xb, w2d_ref[...],
                    preferred_element_type=jnp.float32).astype(jnp.bfloat16)
        hh = jnp.concatenate([h] * _H, axis=1)
        p = g * hh
        out_ref[:, 0:_H] = (
            jnp.dot(p, k_ref[...], preferred_element_type=jnp.float32)
            + jnp.dot(xb, bbr_ref[...], preferred_element_type=jnp.float32)
        )
        out_ref[:, _H:_H + 1] = jnp.ones((_EB, 1), jnp.float32)

    return pl.pallas_call(
        body,
        grid=(grid,),
        in_specs=[
            pl.BlockSpec((_EB, _DIN), lambda i: (i, 0)),
            pl.BlockSpec((_EB, _DE), lambda i: (i, 0)),
            pl.BlockSpec((_DE, _HID), lambda i: (0, 0)),
            pl.BlockSpec((1, _HID), lambda i: (0, 0)),
            pl.BlockSpec((_DIN, _H * _HID), lambda i: (0, 0)),
            pl.BlockSpec((_DIN, _H), lambda i: (0, 0)),
            pl.BlockSpec((_H * _HID, _H), lambda i: (0, 0)),
        ],
        out_specs=pl.BlockSpec((_EB, _W), lambda i: (i, 0)),
        out_shape=jax.ShapeDtypeStruct((_EP, _W), jnp.float32),
    )(xj, ea_bf, Wa, ba, W2d_bf, bbr, kmat)


def _edge_messages2(h1j, ea_bf, Wa, ba, Wb, bb, rmat, k16):
    grid = _E // _EB

    def body(hj_ref, ea_ref, wa_ref, ba_ref, wb_ref, bb_ref, r_ref, k_ref, out_ref):
        h = jax.nn.relu(
            jnp.dot(ea_ref[...], wa_ref[...], preferred_element_type=jnp.float32)
            + ba_ref[...]
        )
        w2 = jnp.dot(h, wb_ref[...], preferred_element_type=jnp.float32) + bb_ref[...]
        rep = jnp.dot(hj_ref[:, 0:_H].astype(jnp.bfloat16), r_ref[...],
                      preferred_element_type=jnp.float32)
        p = (w2 * rep).astype(jnp.bfloat16)
        out_ref[:, 0:_H] = jnp.dot(p, k_ref[...], preferred_element_type=jnp.float32)
        out_ref[:, _H:_H + 1] = jnp.ones((_EB, 1), jnp.float32)

    return pl.pallas_call(
        body,
        grid=(grid,),
        in_specs=[
            pl.BlockSpec((_EB, _W), lambda i: (i, 0)),
            pl.BlockSpec((_EB, _DE), lambda i: (i, 0)),
            pl.BlockSpec((_DE, _HID), lambda i: (0, 0)),
            pl.BlockSpec((1, _HID), lambda i: (0, 0)),
            pl.BlockSpec((_HID, _H * _H), lambda i: (0, 0)),
            pl.BlockSpec((1, _H * _H), lambda i: (0, 0)),
            pl.BlockSpec((_H, _H * _H), lambda i: (0, 0)),
            pl.BlockSpec((_H * _H, _H), lambda i: (0, 0)),
        ],
        out_specs=pl.BlockSpec((_EB, _W), lambda i: (i, 0)),
        out_shape=jax.ShapeDtypeStruct((_EP, _W), jnp.float32),
    )(h1j, ea_bf, Wa, ba, Wb, bb, rmat, k16)


def _node_update(s0, s1, feats, root, bias, wfc=None, bfc=None):
    def body(*refs):
        if wfc is None:
            s0r, s1r, fr, rr, br, out = refs
        else:
            s0r, s1r, fr, rr, br, wr, bwr, out = refs
        ssum = s0r[0:_N, 0:_SW] + s1r[0:_N, 0:_SW]
        cnt = jnp.maximum(ssum[:, _H:_H + 1], 1.0)
        f = fr[...]
        if f.shape[1] > root.shape[0]:
            f = f[:, 0:root.shape[0]]
        h = jax.nn.relu(
            ssum[:, 0:_H] / cnt
            + jnp.dot(f, rr[...], preferred_element_type=jnp.float32)
            + br[...]
        )
        if wfc is None:
            out[:, 0:_H] = h
        else:
            out[...] = (
                jnp.dot(h, wr[...], preferred_element_type=jnp.float32) + bwr[...]
            )

    args = [s0, s1, feats, root, bias]
    if wfc is not None:
        args += [wfc, bfc]
    if wfc is None:
        out_sds = jax.ShapeDtypeStruct((_N, _W), jnp.float32)
    else:
        out_sds = jax.ShapeDtypeStruct((_N, _OUT), jnp.float32)
    return pl.pallas_call(
        body,
        out_shape=out_sds,
    )(*args)


def kernel(x, edge_index, edge_attr, Wa1, ba1, Wb1, bb1, root1, bias1,
           Wa2, ba2, Wb2, bb2, root2, bias2, Wfc, bfc):
    pad = _EP - _E
    src2d = jnp.concatenate(
        [edge_index[0].astype(jnp.int32), jnp.zeros((pad,), jnp.int32)]
    ).reshape(_ROWS, _CHUNK)
    dst2d = jnp.concatenate(
        [edge_index[1].astype(jnp.int32), jnp.full((pad,), _N, jnp.int32)]
    ).reshape(_ROWS, _CHUNK)
    ea_bf = edge_attr.astype(jnp.bfloat16)

    w2d1 = (Wb1.reshape(_HID, _DIN, _H).transpose(1, 2, 0)
            .reshape(_DIN, _H * _HID).astype(jnp.bfloat16))
    bb1r = bb1.reshape(_DIN, _H).astype(jnp.bfloat16)
    wa1_bf = Wa1.astype(jnp.bfloat16)
    wa2_bf = Wa2.astype(jnp.bfloat16)
    kmat = jnp.repeat(jnp.eye(_H, dtype=jnp.bfloat16), _HID, axis=0)
    rmat = jnp.repeat(jnp.eye(_H, dtype=jnp.bfloat16), _H, axis=1)
    k16 = jnp.tile(jnp.eye(_H, dtype=jnp.bfloat16), (_H, 1))
    zeros_n = jnp.zeros((_NP, _W), jnp.float32)

    xj = _gather_rows(x, src2d, _DIN)
    msg1 = _edge_messages1(xj, ea_bf, wa1_bf, ba1.reshape(1, _HID), w2d1, bb1r, kmat)
    s1 = _scatter_add(msg1, dst2d, zeros_n)
    h1 = _node_update(s1[0], s1[1], x, root1, bias1.reshape(1, _H))

    h1j = _gather_rows(h1, src2d, _H)
    msg2 = _edge_messages2(h1j, ea_bf, wa2_bf, ba2.reshape(1, _HID), Wb2,
                           bb2.reshape(1, _H * _H), rmat, k16)
    s2 = _scatter_add(msg2, dst2d, zeros_n)
    out = _node_update(s2[0], s2[1], h1, root2, bias2.reshape(1, _H),
                       Wfc, bfc.reshape(1, _OUT))
    return out
